# Initial kernel scaffold; baseline (speedup 1.0000x reference)
import jax
import jax.numpy as jnp
from jax.experimental import pallas as pl


def kernel(x_user, x_item, PE, rev_user, rev_item, edge_index_u2i, edge_index_i2u, W_pe, b_pe, Wl_u2i_0, Wr_u2i_0, bb_u2i_0, Wl_i2u_0, Wr_i2u_0, bb_i2u_0, ln_g_user_0, ln_b_user_0, ln_g_item_0, ln_b_item_0, Wl_u2i_1, Wr_u2i_1, bb_u2i_1, Wl_i2u_1, Wr_i2u_1, bb_i2u_1, ln_g_user_1, ln_b_user_1, ln_g_item_1, ln_b_item_1):
    raise NotImplementedError("write your pallas kernel here")


# speedup vs baseline: 3.2700x; 3.2700x over previous
import functools

import jax
import jax.numpy as jnp
from jax import lax
from jax.experimental import pallas as pl
from jax.experimental.pallas import tpu as pltpu
from jax.experimental.pallas import tpu_sc as plsc

f32 = jnp.float32

NU = 10000
C = 128
R = 10240
E = 320000
CHUNK = 128
NTS = 16
EP = 323584
TILE_E = EP // NTS
N_CHUNKS = TILE_E // CHUNK
STRIPE = R // NTS
PEC = 128
BR = 2000
GRID = NU // BR

_mesh = plsc.VectorSubcoreMesh(core_axis_name="c", subcore_axis_name="s")


def _sc_prep(rev_u, rev_i, pe, e_u2i, e_i2u, z128, one128):
    @functools.partial(
        pl.kernel,
        out_type=(
            jax.ShapeDtypeStruct((R, PEC), f32),
            jax.ShapeDtypeStruct((R, PEC), f32),
            jax.ShapeDtypeStruct((R, C), f32),
            jax.ShapeDtypeStruct((R, C), f32),
        ),
        mesh=_mesh,
        scratch_types=[
            pltpu.VMEM((CHUNK,), jnp.int32),
            pltpu.VMEM((CHUNK,), jnp.int32),
            pltpu.VMEM((CHUNK, PEC), f32),
            pltpu.VMEM((CHUNK, C), f32),
            pltpu.VMEM_SHARED((R, C), f32),
            pltpu.SemaphoreType.DMA,
        ],
    )
    def k(rev_u_ref, rev_i_ref, pe_ref, eu_ref, ei_ref, z_ref, one_ref,
          peu_out, pei_out, cu_out, ci_out,
          gidx, sidx, rows, ones_v, acc, sem):
        cid = lax.axis_index("c")
        sid = lax.axis_index("s")
        stripe = sid * STRIPE
        pltpu.sync_copy(z_ref, acc.at[pl.ds(stripe, STRIPE)])
        pltpu.sync_copy(one_ref, ones_v)
        plsc.subcore_barrier()

        def work(rev_ref, e_ref, pe_out, c_out):
            def pechunk(j, carry):
                base = stripe + j * CHUNK
                pltpu.sync_copy(rev_ref.at[pl.ds(base, CHUNK)], gidx)
                pltpu.async_copy(pe_ref.at[gidx], rows, sem).wait()
                pltpu.sync_copy(rows, pe_out.at[pl.ds(base, CHUNK)])
                return carry

            lax.fori_loop(0, STRIPE // CHUNK, pechunk, 0)

            tbase = sid * TILE_E

            def cchunk(j, carry):
                base = tbase + j * CHUNK
                pltpu.sync_copy(e_ref.at[1, pl.ds(base, CHUNK)], sidx)
                pltpu.sync_copy(ones_v, acc.at[sidx], add=True)
                return carry

            lax.fori_loop(0, N_CHUNKS, cchunk, 0)
            plsc.subcore_barrier()
            pltpu.sync_copy(acc.at[pl.ds(stripe, STRIPE)],
                            c_out.at[pl.ds(stripe, STRIPE)])

        @pl.when(cid == 0)
        def _():
            work(rev_u_ref, eu_ref, peu_out, cu_out)

        @pl.when(cid == 1)
        def _():
            work(rev_i_ref, ei_ref, pei_out, ci_out)

    return k(rev_u, rev_i, pe, e_u2i, e_i2u, z128, one128)


def _sc_segsum(h_u, h_i, e_u2i, e_i2u, z128):
    @functools.partial(
        pl.kernel,
        out_type=(
            jax.ShapeDtypeStruct((R, C), f32),
            jax.ShapeDtypeStruct((R, C), f32),
        ),
        mesh=_mesh,
        scratch_types=[
            pltpu.VMEM((2, CHUNK), jnp.int32),
            pltpu.VMEM((CHUNK, C), f32),
            pltpu.VMEM_SHARED((R, C), f32),
            pltpu.SemaphoreType.DMA,
        ],
    )
    def k(hu_ref, hi_ref, eu_ref, ei_ref, z_ref, si_out, su_out,
          eidx, rows, acc, sem):
        cid = lax.axis_index("c")
        sid = lax.axis_index("s")
        stripe = sid * STRIPE
        pltpu.sync_copy(z_ref, acc.at[pl.ds(stripe, STRIPE)])
        plsc.subcore_barrier()

        def work(h_ref, e_ref, s_out):
            tbase = sid * TILE_E

            def chunk(j, carry):
                base = tbase + j * CHUNK
                pltpu.sync_copy(e_ref.at[:, pl.ds(base, CHUNK)], eidx)
                pltpu.async_copy(h_ref.at[eidx.at[0]], rows, sem).wait()
                pltpu.sync_copy(rows, acc.at[eidx.at[1]], add=True)
                return carry

            lax.fori_loop(0, N_CHUNKS, chunk, 0)
            plsc.subcore_barrier()
            pltpu.sync_copy(acc.at[pl.ds(stripe, STRIPE)],
                            s_out.at[pl.ds(stripe, STRIPE)])

        @pl.when(cid == 0)
        def _():
            work(hu_ref, eu_ref, si_out)

        @pl.when(cid == 1)
        def _():
            work(hi_ref, ei_ref, su_out)

    return k(h_u, h_i, e_u2i, e_i2u, z128)


def _tc_prep(peu, pei, wpe, bpe, xu, xi):
    def body(peu_ref, pei_ref, wpe_ref, bpe_ref, xu_ref, xi_ref,
             pu_ref, pi_ref, hu_ref, hi_ref):
        w = wpe_ref[...]
        b = bpe_ref[...]
        pu = jnp.maximum(jnp.dot(peu_ref[...], w, preferred_element_type=f32) + b, 0.0)
        pi = jnp.maximum(jnp.dot(pei_ref[...], w, preferred_element_type=f32) + b, 0.0)
        pu_ref[...] = pu
        pi_ref[...] = pi
        hu_ref[...] = xu_ref[...] + pu
        hi_ref[...] = xi_ref[...] + pi

    row = lambda i: (i, 0)
    full = lambda i: (0, 0)
    return pl.pallas_call(
        body,
        grid=(GRID,),
        in_specs=[
            pl.BlockSpec((BR, PEC), row),
            pl.BlockSpec((BR, PEC), row),
            pl.BlockSpec((PEC, C), full),
            pl.BlockSpec((1, C), full),
            pl.BlockSpec((BR, C), row),
            pl.BlockSpec((BR, C), row),
        ],
        out_specs=[pl.BlockSpec((BR, C), row)] * 4,
        out_shape=[jax.ShapeDtypeStruct((NU, C), f32)] * 4,
    )(peu, pei, wpe, bpe, xu, xi)


def _tc_layer(si, su, deg_i, deg_u, hu, hi, pu, pi, params, add_pe):
    (wl_u2i, wr_u2i, bb_u2i, wl_i2u, wr_i2u, bb_i2u,
     g_u, b_u, g_i, b_i) = params

    def body(si_ref, su_ref, di_ref, du_ref, hu_ref, hi_ref, pu_ref, pi_ref,
             wlu_ref, wru_ref, bbu_ref, wli_ref, wri_ref, bbi_ref,
             gu_ref, bu_ref, gi_ref, bi_ref,
             ou_ref, oi_ref):
        def side(s_ref, d_ref, hd_ref, wl_ref, wr_ref, bb_ref,
                 g_ref, b_ref, p_ref, o_ref):
            cnt = d_ref[...][:, 0:1]
            mean = s_ref[...] * (1.0 / jnp.maximum(cnt, 1.0))
            o = (jnp.dot(mean, wl_ref[...], preferred_element_type=f32)
                 + jnp.dot(hd_ref[...], wr_ref[...], preferred_element_type=f32)
                 + bb_ref[...])
            m = jnp.mean(o, axis=-1, keepdims=True)
            v = jnp.mean((o - m) * (o - m), axis=-1, keepdims=True)
            o = (o - m) * lax.rsqrt(v + 1e-5) * g_ref[...] + b_ref[...]
            o = jnp.maximum(o, 0.0)
            if add_pe:
                o = o + p_ref[...]
            o_ref[...] = o

        side(si_ref, di_ref, hi_ref, wlu_ref, wru_ref, bbu_ref,
             gi_ref, bi_ref, pi_ref, oi_ref)
        side(su_ref, du_ref, hu_ref, wli_ref, wri_ref, bbi_ref,
             gu_ref, bu_ref, pu_ref, ou_ref)

    row = lambda i: (i, 0)
    full = lambda i: (0, 0)
    return pl.pallas_call(
        body,
        grid=(GRID,),
        in_specs=[
            pl.BlockSpec((BR, C), row),
            pl.BlockSpec((BR, C), row),
            pl.BlockSpec((BR, C), row),
            pl.BlockSpec((BR, C), row),
            pl.BlockSpec((BR, C), row),
            pl.BlockSpec((BR, C), row),
            pl.BlockSpec((BR, C), row),
            pl.BlockSpec((BR, C), row),
            pl.BlockSpec((C, C), full),
            pl.BlockSpec((C, C), full),
            pl.BlockSpec((1, C), full),
            pl.BlockSpec((C, C), full),
            pl.BlockSpec((C, C), full),
            pl.BlockSpec((1, C), full),
            pl.BlockSpec((1, C), full),
            pl.BlockSpec((1, C), full),
            pl.BlockSpec((1, C), full),
            pl.BlockSpec((1, C), full),
        ],
        out_specs=[pl.BlockSpec((BR, C), row)] * 2,
        out_shape=[jax.ShapeDtypeStruct((NU, C), f32)] * 2,
    )(si, su, deg_i, deg_u, hu, hi, pu, pi,
      wl_u2i, wr_u2i, bb_u2i, wl_i2u, wr_i2u, bb_i2u, g_u, b_u, g_i, b_i)


def kernel(x_user, x_item, PE, rev_user, rev_item, edge_index_u2i,
           edge_index_i2u, W_pe, b_pe, Wl_u2i_0, Wr_u2i_0, bb_u2i_0,
           Wl_i2u_0, Wr_i2u_0, bb_i2u_0, ln_g_user_0, ln_b_user_0,
           ln_g_item_0, ln_b_item_0, Wl_u2i_1, Wr_u2i_1, bb_u2i_1,
           Wl_i2u_1, Wr_i2u_1, bb_i2u_1, ln_g_user_1, ln_b_user_1,
           ln_g_item_1, ln_b_item_1):
    i32 = jnp.int32
    rev_u = jnp.pad(rev_user.astype(i32), (0, R - NU))
    rev_i = jnp.pad(rev_item.astype(i32), (0, R - NU))
    pe = jnp.pad(PE.astype(f32), ((0, 0), (0, PEC - PE.shape[1])))
    wpe = jnp.pad(W_pe.astype(f32), ((0, PEC - W_pe.shape[0]), (0, 0)))

    def pad_e(e):
        e = e.astype(i32)
        pad = EP - e.shape[1]
        src = jnp.pad(e[0], (0, pad))
        dst = jnp.pad(e[1], (0, pad), constant_values=NU)
        return jnp.stack([src, dst])

    eu = pad_e(edge_index_u2i)
    ei = pad_e(edge_index_i2u)
    z128 = jnp.zeros((STRIPE, C), f32)
    one128 = jnp.ones((CHUNK, C), f32)
    bpe = b_pe.reshape(1, C)

    peu, pei, deg_i, deg_u = _sc_prep(rev_u, rev_i, pe, eu, ei, z128, one128)
    pu, pi, hu, hi = _tc_prep(peu, pei, wpe, bpe, x_user, x_item)

    params0 = (Wl_u2i_0, Wr_u2i_0, bb_u2i_0.reshape(1, C),
               Wl_i2u_0, Wr_i2u_0, bb_i2u_0.reshape(1, C),
               ln_g_user_0.reshape(1, C), ln_b_user_0.reshape(1, C),
               ln_g_item_0.reshape(1, C), ln_b_item_0.reshape(1, C))
    params1 = (Wl_u2i_1, Wr_u2i_1, bb_u2i_1.reshape(1, C),
               Wl_i2u_1, Wr_i2u_1, bb_i2u_1.reshape(1, C),
               ln_g_user_1.reshape(1, C), ln_b_user_1.reshape(1, C),
               ln_g_item_1.reshape(1, C), ln_b_item_1.reshape(1, C))

    si0, su0 = _sc_segsum(hu, hi, eu, ei, z128)
    hu1, hi1 = _tc_layer(si0, su0, deg_i, deg_u, hu, hi, pu, pi, params0, True)
    si1, su1 = _sc_segsum(hu1, hi1, eu, ei, z128)
    xu, xi = _tc_layer(si1, su1, deg_i, deg_u, hu1, hi1, pu, pi, params1, False)
    return xu, xi
